# Initial kernel scaffold; baseline (speedup 1.0000x reference)
import jax
import jax.numpy as jnp
from jax.experimental import pallas as pl


def kernel(src, time, mode, emb_loc, emb_mode, emb_hour, emb_min):
    raise NotImplementedError("write your pallas kernel here")


# speedup vs baseline: 10.6855x; 10.6855x over previous
import functools

import jax
import jax.numpy as jnp
from jax import lax
from jax.experimental import pallas as pl
from jax.experimental.pallas import tpu as pltpu
from jax.experimental.pallas import tpu_sc as plsc

EMB = 16
LANES = 16
NUM_CORES = 2
NUM_SUBCORES = 16
NUM_WORKERS = NUM_CORES * NUM_SUBCORES
CHUNK = 2048
COMB = 96 * 8


@functools.cache
def _build(total):
    assert total % (NUM_WORKERS * CHUNK) == 0
    per_w = total // NUM_WORKERS
    n_chunks = per_w // CHUNK
    mesh = plsc.VectorSubcoreMesh(core_axis_name="c", subcore_axis_name="s")

    @functools.partial(
        pl.kernel,
        out_type=jax.ShapeDtypeStruct((total, EMB), jnp.float32),
        mesh=mesh,
        compiler_params=pltpu.CompilerParams(
            needs_layout_passes=False, use_tc_tiling_on_sc=False),
        scratch_types=[
            pltpu.VMEM((CHUNK,), jnp.int32),
            pltpu.VMEM((CHUNK,), jnp.int32),
            pltpu.VMEM((CHUNK,), jnp.int32),
            pltpu.VMEM((CHUNK, EMB), jnp.float32),
            pltpu.VMEM((COMB * EMB,), jnp.float32),
            pltpu.VMEM((24 * EMB,), jnp.float32),
            pltpu.VMEM((4 * EMB,), jnp.float32),
            pltpu.VMEM((8 * EMB,), jnp.float32),
            pltpu.SemaphoreType.DMA,
        ],
    )
    def k(src_hbm, t_hbm, m_hbm, hour_hbm, min_hbm, mode_hbm, loc_hbm,
          out_hbm, idx_v, t_v, m_v, rows_v, comb_v, hour_v, min_v, mode_v,
          sem):
        wid = lax.axis_index("s") * NUM_CORES + lax.axis_index("c")
        base_w = wid * per_w

        pltpu.sync_copy(hour_hbm, hour_v)
        pltpu.sync_copy(min_hbm, min_v)
        pltpu.sync_copy(mode_hbm, mode_v)
        lane = lax.iota(jnp.int32, LANES)

        def build_comb(g, _):
            cvec = lane + g * LANES
            hvec = (cvec >> 5) * EMB
            mivec = ((cvec >> 3) & 3) * EMB
            movec = (cvec & 7) * EMB
            coff = cvec * EMB
            for kk in range(EMB):
                kfull = jnp.full((LANES,), kk, jnp.int32)
                col = (plsc.load_gather(hour_v, [hvec + kfull])
                       + plsc.load_gather(min_v, [mivec + kfull])
                       + plsc.load_gather(mode_v, [movec + kfull]))
                plsc.store_scatter(comb_v, [coff + kfull], col)
            return 0

        lax.fori_loop(0, COMB // LANES, build_comb, 0)

        def chunk_body(g, _):
            base = base_w + g * CHUNK
            pltpu.sync_copy(src_hbm.at[pl.ds(base, CHUNK)], idx_v)
            pltpu.sync_copy(t_hbm.at[pl.ds(base, CHUNK)], t_v)
            pltpu.sync_copy(m_hbm.at[pl.ds(base, CHUNK)], m_v)
            pltpu.async_copy(loc_hbm.at[idx_v], rows_v, sem).wait()

            def group_body(j, _):
                cvec = t_v[pl.ds(j * LANES, LANES)] * 8 + m_v[pl.ds(j * LANES, LANES)]
                coff = cvec * EMB
                rid = lane + j * LANES
                for kk in range(EMB):
                    kfull = jnp.full((LANES,), kk, jnp.int32)
                    col = plsc.load_gather(comb_v, [coff + kfull])
                    plsc.addupdate_scatter(rows_v, [rid, kfull], col)
                return 0

            lax.fori_loop(0, CHUNK // LANES, group_body, 0)
            pltpu.sync_copy(rows_v, out_hbm.at[pl.ds(base, CHUNK)])
            return 0

        lax.fori_loop(0, n_chunks, chunk_body, 0)

    return k


def kernel(src, time, mode, emb_loc, emb_mode, emb_hour, emb_min):
    B, L = src.shape
    src_f = src.reshape(-1).astype(jnp.int32)
    t_f = time.reshape(-1).astype(jnp.int32)
    m_f = mode.reshape(-1).astype(jnp.int32)
    out = _build(B * L)(src_f, t_f, m_f,
                        emb_hour.reshape(-1), emb_min.reshape(-1),
                        emb_mode.reshape(-1), emb_loc)
    return out.reshape(B, L, EMB)


# speedup vs baseline: 13.6015x; 1.2729x over previous
import functools

import jax
import jax.numpy as jnp
from jax import lax
from jax.experimental import pallas as pl
from jax.experimental.pallas import tpu as pltpu
from jax.experimental.pallas import tpu_sc as plsc

EMB = 16
LANES = 16
NUM_CORES = 2
NUM_SUBCORES = 16
NUM_WORKERS = NUM_CORES * NUM_SUBCORES
CHUNK = 2048
COMB = 96 * 8


@functools.cache
def _build(total):
    assert total % (NUM_WORKERS * CHUNK) == 0
    per_w = total // NUM_WORKERS
    n_chunks = per_w // CHUNK
    mesh = plsc.VectorSubcoreMesh(core_axis_name="c", subcore_axis_name="s")

    @functools.partial(
        pl.kernel,
        out_type=jax.ShapeDtypeStruct((total, EMB), jnp.float32),
        mesh=mesh,
        compiler_params=pltpu.CompilerParams(
            needs_layout_passes=False, use_tc_tiling_on_sc=False),
        scratch_types=[
            pltpu.VMEM((CHUNK,), jnp.int32),
            pltpu.VMEM((CHUNK,), jnp.int32),
            pltpu.VMEM((CHUNK,), jnp.int32),
            pltpu.VMEM((CHUNK, EMB), jnp.float32),
            pltpu.VMEM((COMB * EMB,), jnp.float32),
            pltpu.VMEM((24 * EMB,), jnp.float32),
            pltpu.VMEM((4 * EMB,), jnp.float32),
            pltpu.VMEM((8 * EMB,), jnp.float32),
            pltpu.SemaphoreType.DMA,
        ],
    )
    def k(src_hbm, t_hbm, m_hbm, hour_hbm, min_hbm, mode_hbm, loc_hbm,
          out_hbm, idx_v, t_v, m_v, rows_v, comb_v, hour_v, min_v, mode_v,
          sem):
        wid = lax.axis_index("s") * NUM_CORES + lax.axis_index("c")
        base_w = wid * per_w

        pltpu.sync_copy(hour_hbm, hour_v)
        pltpu.sync_copy(min_hbm, min_v)
        pltpu.sync_copy(mode_hbm, mode_v)
        lane = lax.iota(jnp.int32, LANES)

        def build_comb(g, _):
            cvec = lane + g * LANES
            hvec = (cvec >> 5) * EMB
            mivec = ((cvec >> 3) & 3) * EMB
            movec = (cvec & 7) * EMB
            coff = cvec * EMB
            for kk in range(EMB):
                kfull = jnp.full((LANES,), kk, jnp.int32)
                col = (plsc.load_gather(hour_v, [hvec + kfull])
                       + plsc.load_gather(min_v, [mivec + kfull])
                       + plsc.load_gather(mode_v, [movec + kfull]))
                plsc.store_scatter(comb_v, [coff + kfull], col)
            return 0

        lax.fori_loop(0, COMB // LANES, build_comb, 0)

        def chunk_body(g, _):
            base = base_w + g * CHUNK
            pltpu.sync_copy(src_hbm.at[pl.ds(base, CHUNK)], idx_v)
            pltpu.sync_copy(t_hbm.at[pl.ds(base, CHUNK)], t_v)
            pltpu.sync_copy(m_hbm.at[pl.ds(base, CHUNK)], m_v)
            pltpu.async_copy(loc_hbm.at[idx_v], rows_v, sem).wait()

            def group_body(j, _):
                cvec = t_v[pl.ds(j * LANES, LANES)] * 8 + m_v[pl.ds(j * LANES, LANES)]
                coff = cvec * EMB
                rid = lane + j * LANES
                for d in range(EMB):
                    kvec = (lane + d) & (EMB - 1)
                    col = plsc.load_gather(comb_v, [coff + kvec])
                    plsc.addupdate_scatter(rows_v, [rid, kvec], col)
                return 0

            lax.fori_loop(0, CHUNK // LANES, group_body, 0)
            pltpu.sync_copy(rows_v, out_hbm.at[pl.ds(base, CHUNK)])
            return 0

        lax.fori_loop(0, n_chunks, chunk_body, 0)

    return k


def kernel(src, time, mode, emb_loc, emb_mode, emb_hour, emb_min):
    B, L = src.shape
    src_f = src.reshape(-1).astype(jnp.int32)
    t_f = time.reshape(-1).astype(jnp.int32)
    m_f = mode.reshape(-1).astype(jnp.int32)
    out = _build(B * L)(src_f, t_f, m_f,
                        emb_hour.reshape(-1), emb_min.reshape(-1),
                        emb_mode.reshape(-1), emb_loc)
    return out.reshape(B, L, EMB)


# speedup vs baseline: 16.7107x; 1.2286x over previous
import functools

import jax
import jax.numpy as jnp
from jax import lax
from jax.experimental import pallas as pl
from jax.experimental.pallas import tpu as pltpu
from jax.experimental.pallas import tpu_sc as plsc

EMB = 16
LANES = 16
NUM_CORES = 2
NUM_SUBCORES = 16
NUM_WORKERS = NUM_CORES * NUM_SUBCORES
CHUNK = 1024
NBUF = 4
COMB = 96 * 8


@functools.cache
def _build(total):
    assert total % (NUM_WORKERS * CHUNK) == 0
    per_w = total // NUM_WORKERS
    n_chunks = per_w // CHUNK
    assert n_chunks % NBUF == 0 and n_chunks >= 3 * NBUF
    mesh = plsc.VectorSubcoreMesh(core_axis_name="c", subcore_axis_name="s")

    scratch = (
        [pltpu.VMEM((CHUNK,), jnp.int32) for _ in range(NBUF)]
        + [pltpu.VMEM((CHUNK,), jnp.int32) for _ in range(NBUF)]
        + [pltpu.VMEM((CHUNK,), jnp.int32) for _ in range(NBUF)]
        + [pltpu.VMEM((CHUNK, EMB), jnp.float32) for _ in range(NBUF)]
        + [
            pltpu.VMEM((COMB * EMB,), jnp.float32),
            pltpu.VMEM((24 * EMB,), jnp.float32),
            pltpu.VMEM((4 * EMB,), jnp.float32),
            pltpu.VMEM((8 * EMB,), jnp.float32),
        ]
        + [pltpu.SemaphoreType.DMA for _ in range(3 * NBUF)]
    )

    @functools.partial(
        pl.kernel,
        out_type=jax.ShapeDtypeStruct((total, EMB), jnp.float32),
        mesh=mesh,
        compiler_params=pltpu.CompilerParams(
            needs_layout_passes=False, use_tc_tiling_on_sc=False),
        scratch_types=scratch,
    )
    def k(src_hbm, t_hbm, m_hbm, hour_hbm, min_hbm, mode_hbm, loc_hbm,
          out_hbm, *sc):
        idx = sc[0:NBUF]
        tb = sc[NBUF:2 * NBUF]
        mb = sc[2 * NBUF:3 * NBUF]
        rows = sc[3 * NBUF:4 * NBUF]
        comb_v, hour_v, min_v, mode_v = sc[4 * NBUF:4 * NBUF + 4]
        isem = sc[4 * NBUF + 4:4 * NBUF + 4 + NBUF]
        gsem = sc[4 * NBUF + 4 + NBUF:4 * NBUF + 4 + 2 * NBUF]
        osem = sc[4 * NBUF + 4 + 2 * NBUF:]

        wid = lax.axis_index("s") * NUM_CORES + lax.axis_index("c")
        base_w = wid * per_w
        lane = lax.iota(jnp.int32, LANES)
        kvecs = [(lane + d) & (EMB - 1) for d in range(EMB)]

        pltpu.sync_copy(hour_hbm, hour_v)
        pltpu.sync_copy(min_hbm, min_v)
        pltpu.sync_copy(mode_hbm, mode_v)

        def build_comb(g, _):
            cvec = lane + g * LANES
            hoff = (cvec >> 5) * EMB
            mioff = ((cvec >> 3) & 3) * EMB
            mooff = (cvec & 7) * EMB
            coff = cvec * EMB
            for d in range(EMB):
                col = (plsc.load_gather(hour_v, [hoff + kvecs[d]])
                       + plsc.load_gather(min_v, [mioff + kvecs[d]])
                       + plsc.load_gather(mode_v, [mooff + kvecs[d]]))
                plsc.store_scatter(comb_v, [coff + kvecs[d]], col)
            return 0

        lax.fori_loop(0, COMB // LANES, build_comb, 0)

        def stage(j, r):
            base = base_w + j * CHUNK
            pltpu.async_copy(src_hbm.at[pl.ds(base, CHUNK)], idx[r], isem[r])
            pltpu.async_copy(t_hbm.at[pl.ds(base, CHUNK)], tb[r], isem[r])
            pltpu.async_copy(m_hbm.at[pl.ds(base, CHUNK)], mb[r], isem[r])

        def fire(j, r, drain_store):
            base = base_w + j * CHUNK
            pltpu.make_async_copy(src_hbm.at[pl.ds(base, CHUNK)], idx[r], isem[r]).wait()
            pltpu.make_async_copy(t_hbm.at[pl.ds(base, CHUNK)], tb[r], isem[r]).wait()
            pltpu.make_async_copy(m_hbm.at[pl.ds(base, CHUNK)], mb[r], isem[r]).wait()
            if drain_store:
                pbase = base_w + (j - NBUF) * CHUNK
                pltpu.make_async_copy(
                    rows[r], out_hbm.at[pl.ds(pbase, CHUNK)], osem[r]).wait()
            pltpu.async_copy(loc_hbm.at[idx[r]], rows[r], gsem[r])

        def process(j, r):
            base = base_w + j * CHUNK
            pltpu.make_async_copy(loc_hbm.at[idx[r]], rows[r], gsem[r]).wait()

            def group_body(g, _):
                cvec = tb[r][pl.ds(g * LANES, LANES)] * 8 + mb[r][pl.ds(g * LANES, LANES)]
                coff = cvec * EMB
                rid = lane + g * LANES
                cols = [plsc.load_gather(comb_v, [coff + kvecs[d]])
                        for d in range(EMB)]
                for d in range(EMB):
                    plsc.addupdate_scatter(rows[r], [rid, kvecs[d]], cols[d])
                return 0

            lax.fori_loop(0, CHUNK // LANES, group_body, 0)
            pltpu.async_copy(rows[r], out_hbm.at[pl.ds(base, CHUNK)], osem[r])

        stage(0, 0)
        stage(1, 1)
        fire(0, 0, False)
        for i in range(NBUF):
            stage(i + 2, (i + 2) % NBUF)
            fire(i + 1, (i + 1) % NBUF, i + 1 >= NBUF)
            process(i, i % NBUF)

        def block(bk, _):
            i0 = NBUF + bk * NBUF
            for rr in range(NBUF):
                i = i0 + rr
                stage(i + 2, (rr + 2) % NBUF)
                fire(i + 1, (rr + 1) % NBUF, True)
                process(i, rr)
            return 0

        lax.fori_loop(0, (n_chunks - 2 * NBUF) // NBUF, block, 0)

        for i in range(n_chunks - NBUF, n_chunks):
            if i + 2 < n_chunks:
                stage(i + 2, (i + 2) % NBUF)
            if i + 1 < n_chunks:
                fire(i + 1, (i + 1) % NBUF, True)
            process(i, i % NBUF)
        for r in range(NBUF):
            j = n_chunks - NBUF + r
            pltpu.make_async_copy(
                rows[r], out_hbm.at[pl.ds(base_w + j * CHUNK, CHUNK)], osem[r]).wait()

    return k


def kernel(src, time, mode, emb_loc, emb_mode, emb_hour, emb_min):
    B, L = src.shape
    src_f = src.reshape(-1).astype(jnp.int32)
    t_f = time.reshape(-1).astype(jnp.int32)
    m_f = mode.reshape(-1).astype(jnp.int32)
    out = _build(B * L)(src_f, t_f, m_f,
                        emb_hour.reshape(-1), emb_min.reshape(-1),
                        emb_mode.reshape(-1), emb_loc)
    return out.reshape(B, L, EMB)
